# baseline (speedup 1.0000x reference)
import functools

import jax
import jax.numpy as jnp
from jax import lax
from jax.experimental import pallas as pl
from jax.experimental.pallas import tpu as pltpu
from jax.experimental.pallas import tpu_sc as plsc

N, E, D, A, H = 10000, 320000, 128, 16, 16
DELTA_T = 0.1

NC, NS = 1, 16
NW = NC * NS
CH = 128
EPT = 20480
EP = NW * EPT
NCHUNK = EPT // CH
NP = 10240
RPT = NP // NS


_BR = 2048
_R8 = EP // 8


def _mlp_body(attr, w1x, b1x, w2x, b2x, w1z, b1z, w2z, b2z, out):
    a = attr[...]
    hx = jnp.maximum(jnp.dot(a, w1x[...], preferred_element_type=jnp.float32)
                     + b1x[...], 0.0)
    dx = jax.nn.softplus(jnp.dot(hx, w2x[...],
                                 preferred_element_type=jnp.float32)
                         + b2x[...]) + 1e-6
    hz = jnp.maximum(jnp.dot(a, w1z[...], preferred_element_type=jnp.float32)
                     + b1z[...], 0.0)
    dz = jax.nn.softplus(jnp.dot(hz, w2z[...],
                                 preferred_element_type=jnp.float32)
                         + b2z[...]) + 1e-6
    denom = jnp.maximum(jnp.abs(dx) + jnp.abs(dz), 1e-6)
    w = dz / denom
    row = (pl.program_id(0) * _BR
           + jax.lax.broadcasted_iota(jnp.int32, (_BR, 8), 0))
    out[...] = jnp.where(row < E // 8, w, 0.0)


def _edge_w(edge_attr, W1_dx, b1_dx, W2_dx, b2_dx, W1_dz, b1_dz, W2_dz, b2_dz):
    attr8 = edge_attr.reshape(E // 8, 8 * A)
    attr8 = jnp.concatenate(
        [attr8, jnp.zeros((_R8 - E // 8, 8 * A), jnp.float32)])
    eye8 = jnp.eye(8, dtype=jnp.float32)
    w1x = jnp.kron(eye8, W1_dx); b1x = jnp.tile(b1_dx, 8)
    w2x = jnp.kron(eye8, W2_dx); b2x = jnp.tile(b2_dx, 8)
    w1z = jnp.kron(eye8, W1_dz); b1z = jnp.tile(b1_dz, 8)
    w2z = jnp.kron(eye8, W2_dz); b2z = jnp.tile(b2_dz, 8)
    full = lambda s: pl.BlockSpec(s, lambda i: (0,) * len(s))
    wp = pl.pallas_call(
        _mlp_body,
        grid=(_R8 // _BR,),
        in_specs=[
            pl.BlockSpec((_BR, 8 * A), lambda i: (i, 0)),
            full((8 * A, 8 * A)), full((8 * A,)), full((8 * A, 8)), full((8,)),
            full((8 * A, 8 * A)), full((8 * A,)), full((8 * A, 8)), full((8,)),
        ],
        out_specs=pl.BlockSpec((_BR, 8), lambda i: (i, 0)),
        out_shape=jax.ShapeDtypeStruct((_R8, 8), jnp.float32),
    )(attr8, w1x, b1x, w2x, b2x, w1z, b1z, w2z, b2z)
    return wp.reshape(EP)


_BN = 1000


def _u2_body(u, w, out):
    out[...] = jnp.tanh(jnp.dot(u[...], w[...],
                                preferred_element_type=jnp.float32))


def _u2_tc(u, W):
    return pl.pallas_call(
        _u2_body,
        grid=(N // _BN,),
        in_specs=[pl.BlockSpec((_BN, D), lambda i: (i, 0)),
                  pl.BlockSpec((D, D), lambda i: (0, 0))],
        out_specs=pl.BlockSpec((_BN, D), lambda i: (i, 0)),
        out_shape=jax.ShapeDtypeStruct((N, D), jnp.float32),
    )(u, W)


QCH = 16
NQ = NCHUNK // QCH


def _sc_body(u2_hbm, dsts_hbm, srcs_hbm, ws_hbm, duA_hbm, s_hbm,
             dst_q, src_q, w_q, rows0, rows1,
             gsem0, gsem1, ssem0, ssem1, du_sh, s_sh):
    cid = lax.axis_index("c")
    sid = lax.axis_index("s")
    wid = sid * NC + cid
    rows = (rows0, rows1)
    gsem = (gsem0, gsem1)
    ssem = (ssem0, ssem1)

    def zrow(i, _):
        for c in range(D // 16):
            rows0[i, pl.ds(c * 16, 16)] = jnp.zeros((16,), jnp.float32)
        return 0
    lax.fori_loop(0, CH, zrow, 0, unroll=4)

    base = sid * RPT
    def zdu(k, _):
        pltpu.sync_copy(rows0, du_sh.at[pl.ds(base + k * CH, CH)])
        pltpu.sync_copy(rows0.at[0], s_sh.at[pl.ds(base + k * CH, CH)])
        return 0
    lax.fori_loop(0, RPT // CH, zdu, 0)

    plsc.subcore_barrier()

    def scale(c, rbuf):
        def grp(g, _):
            w16 = w_q[c, pl.ds(g * 16, 16)]
            for l in range(16):
                wb = lax.broadcast_in_dim(w16[l], (16,), ())
                i = g * 16 + l
                for k in range(D // 16):
                    sl = pl.ds(k * 16, 16)
                    rbuf[i, sl] = rbuf[i, sl] * wb
            return 0
        lax.fori_loop(0, CH // 16, grp, 0)

    for q in range(NQ):
        pltpu.sync_copy(dsts_hbm.at[wid, pl.ds(q * QCH, QCH)], dst_q)
        pltpu.sync_copy(srcs_hbm.at[wid, pl.ds(q * QCH, QCH)], src_q)
        pltpu.sync_copy(ws_hbm.at[wid, pl.ds(q * QCH, QCH)], w_q)
        pltpu.async_copy(u2_hbm.at[dst_q.at[0]], rows0, gsem0)

        def pair(j2, _):
            for p in range(2):
                c = 2 * j2 + p
                pltpu.make_async_copy(
                    u2_hbm.at[dst_q.at[c]], rows[p], gsem[p]).wait()

                @pl.when(c < QCH - 1)
                def _():
                    @pl.when(c >= 1)
                    def _():
                        pltpu.make_async_copy(
                            rows[1 - p], du_sh.at[src_q.at[c]],
                            ssem[1 - p]).wait()
                    pltpu.async_copy(
                        u2_hbm.at[dst_q.at[c + 1]], rows[1 - p], gsem[1 - p])

                scale(c, rows[p])
                pltpu.async_copy(rows[p], du_sh.at[src_q.at[c]], ssem[p],
                                 add=True)
                pltpu.sync_copy(w_q.at[c], s_sh.at[src_q.at[c]], add=True)
            return 0
        lax.fori_loop(0, QCH // 2, pair, 0)

        pltpu.make_async_copy(rows0, du_sh.at[src_q.at[0]], ssem0).wait()
        pltpu.make_async_copy(rows1, du_sh.at[src_q.at[0]], ssem1).wait()

    plsc.subcore_barrier()

    pltpu.sync_copy(du_sh.at[pl.ds(base, RPT)], duA_hbm.at[cid, pl.ds(base, RPT)])
    @pl.when(sid == 0)
    def _():
        pltpu.sync_copy(s_sh, s_hbm.at[cid])


def _sc_scatter(u2, dsts, srcs, ws):
    mesh = plsc.VectorSubcoreMesh(core_axis_name="c", subcore_axis_name="s",
                                  num_cores=NC, num_subcores=NS)
    f = pl.kernel(
        _sc_body,
        out_type=[jax.ShapeDtypeStruct((NC, NP, D), jnp.float32),
                  jax.ShapeDtypeStruct((NC, NP), jnp.float32)],
        mesh=mesh,
        scratch_types=[
            pltpu.VMEM((QCH, CH), jnp.int32),
            pltpu.VMEM((QCH, CH), jnp.int32),
            pltpu.VMEM((QCH, CH), jnp.float32),
            pltpu.VMEM((CH, D), jnp.float32),
            pltpu.VMEM((CH, D), jnp.float32),
            pltpu.SemaphoreType.DMA,
            pltpu.SemaphoreType.DMA,
            pltpu.SemaphoreType.DMA,
            pltpu.SemaphoreType.DMA,
            pltpu.VMEM_SHARED((NP, D), jnp.float32),
            pltpu.VMEM_SHARED((NP,), jnp.float32),
        ],
    )
    return f(u2, dsts, srcs, ws)


def _final_body(u2, duA, s, out):
    du = jnp.sum(duA[...], axis=0)
    stot = jnp.sum(s[...], axis=0)
    v = du - stot * u2[...]
    out[...] = u2[...] + jnp.tanh(DELTA_T * v)


def _final_tc(u2, duA, s):
    return pl.pallas_call(
        _final_body,
        grid=(N // _BN,),
        in_specs=[pl.BlockSpec((_BN, D), lambda i: (i, 0)),
                  pl.BlockSpec((NC, _BN, D), lambda i: (0, i, 0)),
                  pl.BlockSpec((NC, _BN, 1), lambda i: (0, i, 0))],
        out_specs=pl.BlockSpec((_BN, D), lambda i: (i, 0)),
        out_shape=jax.ShapeDtypeStruct((N, D), jnp.float32),
    )(u2, duA, s.reshape(NC, NP, 1))


def kernel(u, edge_index, edge_attr, W, W1_dx, b1_dx, W2_dx, b2_dx,
           W1_dz, b1_dz, W2_dz, b2_dz):
    wp = _edge_w(edge_attr, W1_dx, b1_dx, W2_dx, b2_dx,
                 W1_dz, b1_dz, W2_dz, b2_dz)
    u2 = _u2_tc(u, W)

    pad = EP - E
    ei = jnp.concatenate(
        [edge_index, jnp.zeros((2, pad), jnp.int32)], axis=1)
    srcs = ei[0].reshape(NW, NCHUNK, CH)
    dsts = ei[1].reshape(NW, NCHUNK, CH)
    ws = wp.reshape(NW, NCHUNK, CH)

    duA, s = _sc_scatter(u2, dsts, srcs, ws)
    return _final_tc(u2, duA, s)


# speedup vs baseline: 1.1046x; 1.1046x over previous
import functools

import jax
import jax.numpy as jnp
from jax import lax
from jax.experimental import pallas as pl
from jax.experimental.pallas import tpu as pltpu
from jax.experimental.pallas import tpu_sc as plsc

N, E, D, A, H = 10000, 320000, 128, 16, 16
DELTA_T = 0.1

NC, NS = 2, 16
NW = NC * NS
CH = 128
EPT = 10240
EP = NW * EPT
NCHUNK = EPT // CH
NP = 10240
RPT = NP // NS


_BR = 2048
_R8 = EP // 8


def _mlp_body(attr, w1x, b1x, w2x, b2x, w1z, b1z, w2z, b2z, out):
    a = attr[...]
    hx = jnp.maximum(jnp.dot(a, w1x[...], preferred_element_type=jnp.float32)
                     + b1x[...], 0.0)
    dx = jax.nn.softplus(jnp.dot(hx, w2x[...],
                                 preferred_element_type=jnp.float32)
                         + b2x[...]) + 1e-6
    hz = jnp.maximum(jnp.dot(a, w1z[...], preferred_element_type=jnp.float32)
                     + b1z[...], 0.0)
    dz = jax.nn.softplus(jnp.dot(hz, w2z[...],
                                 preferred_element_type=jnp.float32)
                         + b2z[...]) + 1e-6
    denom = jnp.maximum(jnp.abs(dx) + jnp.abs(dz), 1e-6)
    w = dz / denom
    row = (pl.program_id(0) * _BR
           + jax.lax.broadcasted_iota(jnp.int32, (_BR, 8), 0))
    out[...] = jnp.where(row < E // 8, w, 0.0)


def _edge_w(edge_attr, W1_dx, b1_dx, W2_dx, b2_dx, W1_dz, b1_dz, W2_dz, b2_dz):
    attr8 = edge_attr.reshape(E // 8, 8 * A)
    attr8 = jnp.concatenate(
        [attr8, jnp.zeros((_R8 - E // 8, 8 * A), jnp.float32)])
    eye8 = jnp.eye(8, dtype=jnp.float32)
    w1x = jnp.kron(eye8, W1_dx); b1x = jnp.tile(b1_dx, 8)
    w2x = jnp.kron(eye8, W2_dx); b2x = jnp.tile(b2_dx, 8)
    w1z = jnp.kron(eye8, W1_dz); b1z = jnp.tile(b1_dz, 8)
    w2z = jnp.kron(eye8, W2_dz); b2z = jnp.tile(b2_dz, 8)
    full = lambda s: pl.BlockSpec(s, lambda i: (0,) * len(s))
    wp = pl.pallas_call(
        _mlp_body,
        grid=(_R8 // _BR,),
        in_specs=[
            pl.BlockSpec((_BR, 8 * A), lambda i: (i, 0)),
            full((8 * A, 8 * A)), full((8 * A,)), full((8 * A, 8)), full((8,)),
            full((8 * A, 8 * A)), full((8 * A,)), full((8 * A, 8)), full((8,)),
        ],
        out_specs=pl.BlockSpec((_BR, 8), lambda i: (i, 0)),
        out_shape=jax.ShapeDtypeStruct((_R8, 8), jnp.float32),
    )(attr8, w1x, b1x, w2x, b2x, w1z, b1z, w2z, b2z)
    return wp.reshape(EP)


_BN = 1000


def _u2_body(u, w, out):
    out[...] = jnp.tanh(jnp.dot(u[...], w[...],
                                preferred_element_type=jnp.float32))


def _u2_tc(u, W):
    return pl.pallas_call(
        _u2_body,
        grid=(N // _BN,),
        in_specs=[pl.BlockSpec((_BN, D), lambda i: (i, 0)),
                  pl.BlockSpec((D, D), lambda i: (0, 0))],
        out_specs=pl.BlockSpec((_BN, D), lambda i: (i, 0)),
        out_shape=jax.ShapeDtypeStruct((N, D), jnp.float32),
    )(u, W)


QCH = 8
CK0 = 40
CK1 = 2 * NCHUNK - CK0
TOT_CK = 2 * NCHUNK


def _sc_body(u2_hbm, dsts_hbm, srcs_hbm, ws_hbm, duA_hbm, s_hbm,
             dst_q, src_q, w_q, rows0, rows1,
             gsem0, gsem1, ssem0, ssem1, du_sh, s_sh):
    cid = lax.axis_index("c")
    sid = lax.axis_index("s")
    wid = sid * NC + cid
    rows = (rows0, rows1)
    gsem = (gsem0, gsem1)
    ssem = (ssem0, ssem1)

    def zrow(i, _):
        for c in range(D // 16):
            rows0[i, pl.ds(c * 16, 16)] = jnp.zeros((16,), jnp.float32)
        return 0
    lax.fori_loop(0, CH, zrow, 0, unroll=4)

    base = sid * RPT
    def zdu(k, _):
        pltpu.sync_copy(rows0, du_sh.at[pl.ds(base + k * CH, CH)])
        pltpu.sync_copy(rows0.at[0], s_sh.at[pl.ds(base + k * CH, CH)])
        return 0
    lax.fori_loop(0, RPT // CH, zdu, 0)

    plsc.subcore_barrier()

    def scale(c, rbuf):
        def grp(g, _):
            w16 = w_q[c, pl.ds(g * 16, 16)]
            for l in range(16):
                wb = lax.broadcast_in_dim(w16[l], (16,), ())
                i = g * 16 + l
                for k in range(D // 16):
                    sl = pl.ds(k * 16, 16)
                    rbuf[i, sl] = rbuf[i, sl] * wb
            return 0
        lax.fori_loop(0, CH // 16, grp, 0)

    start = sid * TOT_CK + jnp.where(cid == 0, 0, CK0)
    nq = jnp.where(cid == 0, CK0 // QCH, CK1 // QCH)

    def round_body(q, _):
        off = start + q * QCH
        pltpu.sync_copy(dsts_hbm.at[pl.ds(off, QCH)], dst_q)
        pltpu.sync_copy(srcs_hbm.at[pl.ds(off, QCH)], src_q)
        pltpu.sync_copy(ws_hbm.at[pl.ds(off, QCH)], w_q)
        pltpu.async_copy(u2_hbm.at[dst_q.at[0]], rows0, gsem0)

        def pair(j2, _):
            for p in range(2):
                c = 2 * j2 + p
                pltpu.make_async_copy(
                    u2_hbm.at[dst_q.at[c]], rows[p], gsem[p]).wait()

                @pl.when(c < QCH - 1)
                def _():
                    @pl.when(c >= 1)
                    def _():
                        pltpu.make_async_copy(
                            rows[1 - p], du_sh.at[src_q.at[c]],
                            ssem[1 - p]).wait()
                    pltpu.async_copy(
                        u2_hbm.at[dst_q.at[c + 1]], rows[1 - p], gsem[1 - p])

                scale(c, rows[p])
                pltpu.async_copy(rows[p], du_sh.at[src_q.at[c]], ssem[p],
                                 add=True)
                pltpu.sync_copy(w_q.at[c], s_sh.at[src_q.at[c]], add=True)
            return 0
        lax.fori_loop(0, QCH // 2, pair, 0)

        pltpu.make_async_copy(rows0, du_sh.at[src_q.at[0]], ssem0).wait()
        pltpu.make_async_copy(rows1, du_sh.at[src_q.at[0]], ssem1).wait()
        return 0
    lax.fori_loop(0, nq, round_body, 0)

    plsc.subcore_barrier()

    pltpu.sync_copy(du_sh.at[pl.ds(base, RPT)], duA_hbm.at[cid, pl.ds(base, RPT)])
    @pl.when(sid == 0)
    def _():
        pltpu.sync_copy(s_sh, s_hbm.at[cid])


def _sc_scatter(u2, dsts, srcs, ws):
    mesh = plsc.VectorSubcoreMesh(core_axis_name="c", subcore_axis_name="s",
                                  num_cores=NC, num_subcores=NS)
    f = pl.kernel(
        _sc_body,
        out_type=[jax.ShapeDtypeStruct((NC, NP, D), jnp.float32),
                  jax.ShapeDtypeStruct((NC, NP), jnp.float32)],
        mesh=mesh,
        scratch_types=[
            pltpu.VMEM((QCH, CH), jnp.int32),
            pltpu.VMEM((QCH, CH), jnp.int32),
            pltpu.VMEM((QCH, CH), jnp.float32),
            pltpu.VMEM((CH, D), jnp.float32),
            pltpu.VMEM((CH, D), jnp.float32),
            pltpu.SemaphoreType.DMA,
            pltpu.SemaphoreType.DMA,
            pltpu.SemaphoreType.DMA,
            pltpu.SemaphoreType.DMA,
            pltpu.VMEM_SHARED((NP, D), jnp.float32),
            pltpu.VMEM_SHARED((NP,), jnp.float32),
        ],
    )
    return f(u2, dsts, srcs, ws)


def _final_body(u2, duA, s, out):
    du = jnp.sum(duA[...], axis=0)
    stot = jnp.sum(s[...], axis=0)
    v = du - stot * u2[...]
    out[...] = u2[...] + jnp.tanh(DELTA_T * v)


def _final_tc(u2, duA, s):
    return pl.pallas_call(
        _final_body,
        grid=(N // _BN,),
        in_specs=[pl.BlockSpec((_BN, D), lambda i: (i, 0)),
                  pl.BlockSpec((NC, _BN, D), lambda i: (0, i, 0)),
                  pl.BlockSpec((NC, _BN, 1), lambda i: (0, i, 0))],
        out_specs=pl.BlockSpec((_BN, D), lambda i: (i, 0)),
        out_shape=jax.ShapeDtypeStruct((N, D), jnp.float32),
    )(u2, duA, s.reshape(NC, NP, 1))


def kernel(u, edge_index, edge_attr, W, W1_dx, b1_dx, W2_dx, b2_dx,
           W1_dz, b1_dz, W2_dz, b2_dz):
    wp = _edge_w(edge_attr, W1_dx, b1_dx, W2_dx, b2_dx,
                 W1_dz, b1_dz, W2_dz, b2_dz)
    u2 = _u2_tc(u, W)

    pad = EP - E
    ei = jnp.concatenate(
        [edge_index, jnp.zeros((2, pad), jnp.int32)], axis=1)
    srcs = ei[0].reshape(EP // CH, CH)
    dsts = ei[1].reshape(EP // CH, CH)
    ws = wp.reshape(EP // CH, CH)

    duA, s = _sc_scatter(u2, dsts, srcs, ws)
    return _final_tc(u2, duA, s)


# speedup vs baseline: 1.3324x; 1.2062x over previous
import functools

import jax
import jax.numpy as jnp
from jax import lax
from jax.experimental import pallas as pl
from jax.experimental.pallas import tpu as pltpu
from jax.experimental.pallas import tpu_sc as plsc

N, E, D, A, H = 10000, 320000, 128, 16, 16
DELTA_T = 0.1

NC, NS = 2, 16
NW = NC * NS
CH = 128
EPT = 10240
EP = NW * EPT
NCHUNK = EPT // CH
NP = 10240
RPT = NP // NS


_BR = 2048
_R8 = EP // 8


def _mlp_body(attr, w1x, b1x, w2x, b2x, w1z, b1z, w2z, b2z, out):
    a = attr[...]
    hx = jnp.maximum(jnp.dot(a, w1x[...], preferred_element_type=jnp.float32)
                     + b1x[...], 0.0)
    dx = jax.nn.softplus(jnp.dot(hx, w2x[...],
                                 preferred_element_type=jnp.float32)
                         + b2x[...]) + 1e-6
    hz = jnp.maximum(jnp.dot(a, w1z[...], preferred_element_type=jnp.float32)
                     + b1z[...], 0.0)
    dz = jax.nn.softplus(jnp.dot(hz, w2z[...],
                                 preferred_element_type=jnp.float32)
                         + b2z[...]) + 1e-6
    denom = jnp.maximum(jnp.abs(dx) + jnp.abs(dz), 1e-6)
    w = dz / denom
    row = (pl.program_id(0) * _BR
           + jax.lax.broadcasted_iota(jnp.int32, (_BR, 8), 0))
    out[...] = jnp.where(row < E // 8, w, 0.0)


def _edge_w(edge_attr, W1_dx, b1_dx, W2_dx, b2_dx, W1_dz, b1_dz, W2_dz, b2_dz):
    attr8 = edge_attr.reshape(E // 8, 8 * A)
    attr8 = jnp.concatenate(
        [attr8, jnp.zeros((_R8 - E // 8, 8 * A), jnp.float32)])
    eye8 = jnp.eye(8, dtype=jnp.float32)
    w1x = jnp.kron(eye8, W1_dx); b1x = jnp.tile(b1_dx, 8)
    w2x = jnp.kron(eye8, W2_dx); b2x = jnp.tile(b2_dx, 8)
    w1z = jnp.kron(eye8, W1_dz); b1z = jnp.tile(b1_dz, 8)
    w2z = jnp.kron(eye8, W2_dz); b2z = jnp.tile(b2_dz, 8)
    full = lambda s: pl.BlockSpec(s, lambda i: (0,) * len(s))
    wp = pl.pallas_call(
        _mlp_body,
        grid=(_R8 // _BR,),
        in_specs=[
            pl.BlockSpec((_BR, 8 * A), lambda i: (i, 0)),
            full((8 * A, 8 * A)), full((8 * A,)), full((8 * A, 8)), full((8,)),
            full((8 * A, 8 * A)), full((8 * A,)), full((8 * A, 8)), full((8,)),
        ],
        out_specs=pl.BlockSpec((_BR, 8), lambda i: (i, 0)),
        out_shape=jax.ShapeDtypeStruct((_R8, 8), jnp.float32),
    )(attr8, w1x, b1x, w2x, b2x, w1z, b1z, w2z, b2z)
    return wp.reshape(EP)


_BN = 1000


def _u2_body(u, w, out):
    out[...] = jnp.tanh(jnp.dot(u[...], w[...],
                                preferred_element_type=jnp.float32))


def _u2_tc(u, W):
    return pl.pallas_call(
        _u2_body,
        grid=(N // _BN,),
        in_specs=[pl.BlockSpec((_BN, D), lambda i: (i, 0)),
                  pl.BlockSpec((D, D), lambda i: (0, 0))],
        out_specs=pl.BlockSpec((_BN, D), lambda i: (i, 0)),
        out_shape=jax.ShapeDtypeStruct((N, D), jnp.float32),
    )(u, W)


QCH = 8
CK0 = 120
CK1 = 2 * NCHUNK - CK0
TOT_CK = 2 * NCHUNK


def _sc_body(u2_hbm, dsts_hbm, srcs_hbm, ws_hbm, duA_hbm, s_hbm,
             dst_q, src_q, w_q, rows0, rows1,
             gsem0, gsem1, ssem0, ssem1, du_sh, s_sh):
    cid = lax.axis_index("c")
    sid = lax.axis_index("s")
    wid = sid * NC + cid
    rows = (rows0, rows1)
    gsem = (gsem0, gsem1)
    ssem = (ssem0, ssem1)

    def zrow(i, _):
        for c in range(D // 16):
            rows0[i, pl.ds(c * 16, 16)] = jnp.zeros((16,), jnp.float32)
        return 0
    lax.fori_loop(0, CH, zrow, 0, unroll=4)

    base = sid * RPT
    def zdu(k, _):
        pltpu.sync_copy(rows0, du_sh.at[pl.ds(base + k * CH, CH)])
        pltpu.sync_copy(rows0.at[0], s_sh.at[pl.ds(base + k * CH, CH)])
        return 0
    lax.fori_loop(0, RPT // CH, zdu, 0)

    plsc.subcore_barrier()

    def scale(c, rbuf):
        def grp(g, _):
            w16 = w_q[c, pl.ds(g * 16, 16)]
            for l in range(16):
                wb = lax.broadcast_in_dim(w16[l], (16,), ())
                i = g * 16 + l
                for k in range(D // 16):
                    sl = pl.ds(k * 16, 16)
                    rbuf[i, sl] = rbuf[i, sl] * wb
            return 0
        lax.fori_loop(0, CH // 16, grp, 0)

    start = sid * TOT_CK + jnp.where(cid == 0, 0, CK0)
    nq = jnp.where(cid == 0, CK0 // QCH, CK1 // QCH)

    def round_body(q, _):
        off = start + q * QCH
        pltpu.sync_copy(dsts_hbm.at[pl.ds(off, QCH)], dst_q)
        pltpu.sync_copy(srcs_hbm.at[pl.ds(off, QCH)], src_q)
        pltpu.sync_copy(ws_hbm.at[pl.ds(off, QCH)], w_q)
        pltpu.async_copy(u2_hbm.at[dst_q.at[0]], rows0, gsem0)

        def pair(j2, _):
            for p in range(2):
                c = 2 * j2 + p
                pltpu.make_async_copy(
                    u2_hbm.at[dst_q.at[c]], rows[p], gsem[p]).wait()

                @pl.when(c < QCH - 1)
                def _():
                    @pl.when(c >= 1)
                    def _():
                        pltpu.make_async_copy(
                            rows[1 - p], du_sh.at[src_q.at[c]],
                            ssem[1 - p]).wait()
                    pltpu.async_copy(
                        u2_hbm.at[dst_q.at[c + 1]], rows[1 - p], gsem[1 - p])

                scale(c, rows[p])
                pltpu.async_copy(rows[p], du_sh.at[src_q.at[c]], ssem[p],
                                 add=True)
                pltpu.sync_copy(w_q.at[c], s_sh.at[src_q.at[c]], add=True)
            return 0
        lax.fori_loop(0, QCH // 2, pair, 0)

        pltpu.make_async_copy(rows0, du_sh.at[src_q.at[0]], ssem0).wait()
        pltpu.make_async_copy(rows1, du_sh.at[src_q.at[0]], ssem1).wait()
        return 0
    lax.fori_loop(0, nq, round_body, 0)

    plsc.subcore_barrier()

    pltpu.sync_copy(du_sh.at[pl.ds(base, RPT)], duA_hbm.at[cid, pl.ds(base, RPT)])
    @pl.when(sid == 0)
    def _():
        pltpu.sync_copy(s_sh, s_hbm.at[cid])


def _sc_scatter(u2, dsts, srcs, ws):
    mesh = plsc.VectorSubcoreMesh(core_axis_name="c", subcore_axis_name="s",
                                  num_cores=NC, num_subcores=NS)
    f = pl.kernel(
        _sc_body,
        out_type=[jax.ShapeDtypeStruct((NC, NP, D), jnp.float32),
                  jax.ShapeDtypeStruct((NC, NP), jnp.float32)],
        mesh=mesh,
        scratch_types=[
            pltpu.VMEM((QCH, CH), jnp.int32),
            pltpu.VMEM((QCH, CH), jnp.int32),
            pltpu.VMEM((QCH, CH), jnp.float32),
            pltpu.VMEM((CH, D), jnp.float32),
            pltpu.VMEM((CH, D), jnp.float32),
            pltpu.SemaphoreType.DMA,
            pltpu.SemaphoreType.DMA,
            pltpu.SemaphoreType.DMA,
            pltpu.SemaphoreType.DMA,
            pltpu.VMEM_SHARED((NP, D), jnp.float32),
            pltpu.VMEM_SHARED((NP,), jnp.float32),
        ],
    )
    return f(u2, dsts, srcs, ws)


def _final_body(u2, duA, s, out):
    du = jnp.sum(duA[...], axis=0)
    stot = jnp.sum(s[...], axis=0)
    v = du - stot * u2[...]
    out[...] = u2[...] + jnp.tanh(DELTA_T * v)


def _final_tc(u2, duA, s):
    return pl.pallas_call(
        _final_body,
        grid=(N // _BN,),
        in_specs=[pl.BlockSpec((_BN, D), lambda i: (i, 0)),
                  pl.BlockSpec((NC, _BN, D), lambda i: (0, i, 0)),
                  pl.BlockSpec((NC, _BN, 1), lambda i: (0, i, 0))],
        out_specs=pl.BlockSpec((_BN, D), lambda i: (i, 0)),
        out_shape=jax.ShapeDtypeStruct((N, D), jnp.float32),
    )(u2, duA, s.reshape(NC, NP, 1))


def kernel(u, edge_index, edge_attr, W, W1_dx, b1_dx, W2_dx, b2_dx,
           W1_dz, b1_dz, W2_dz, b2_dz):
    wp = _edge_w(edge_attr, W1_dx, b1_dx, W2_dx, b2_dx,
                 W1_dz, b1_dz, W2_dz, b2_dz)
    u2 = _u2_tc(u, W)

    pad = EP - E
    ei = jnp.concatenate(
        [edge_index, jnp.zeros((2, pad), jnp.int32)], axis=1)
    srcs = ei[0].reshape(EP // CH, CH)
    dsts = ei[1].reshape(EP // CH, CH)
    ws = wp.reshape(EP // CH, CH)

    duA, s = _sc_scatter(u2, dsts, srcs, ws)
    return _final_tc(u2, duA, s)


# speedup vs baseline: 1.3336x; 1.0009x over previous
import functools

import jax
import jax.numpy as jnp
from jax import lax
from jax.experimental import pallas as pl
from jax.experimental.pallas import tpu as pltpu
from jax.experimental.pallas import tpu_sc as plsc

N, E, D, A, H = 10000, 320000, 128, 16, 16
DELTA_T = 0.1

NC, NS = 2, 16
NW = NC * NS
CH = 128
EPT = 10240
EP = NW * EPT
NCHUNK = EPT // CH
NP = 10240
RPT = NP // NS


_BR = 2048
_R8 = EP // 8


def _mlp_body(attr, w1x, b1x, w2x, b2x, w1z, b1z, w2z, b2z, out):
    a = attr[...]
    hx = jnp.maximum(jnp.dot(a, w1x[...], preferred_element_type=jnp.float32)
                     + b1x[...], 0.0)
    dx = jax.nn.softplus(jnp.dot(hx, w2x[...],
                                 preferred_element_type=jnp.float32)
                         + b2x[...]) + 1e-6
    hz = jnp.maximum(jnp.dot(a, w1z[...], preferred_element_type=jnp.float32)
                     + b1z[...], 0.0)
    dz = jax.nn.softplus(jnp.dot(hz, w2z[...],
                                 preferred_element_type=jnp.float32)
                         + b2z[...]) + 1e-6
    denom = jnp.maximum(jnp.abs(dx) + jnp.abs(dz), 1e-6)
    w = dz / denom
    row = (pl.program_id(0) * _BR
           + jax.lax.broadcasted_iota(jnp.int32, (_BR, 8), 0))
    out[...] = jnp.where(row < E // 8, w, 0.0)


def _edge_w(edge_attr, W1_dx, b1_dx, W2_dx, b2_dx, W1_dz, b1_dz, W2_dz, b2_dz):
    attr8 = edge_attr.reshape(E // 8, 8 * A)
    attr8 = jnp.concatenate(
        [attr8, jnp.zeros((_R8 - E // 8, 8 * A), jnp.float32)])
    eye8 = jnp.eye(8, dtype=jnp.float32)
    w1x = jnp.kron(eye8, W1_dx); b1x = jnp.tile(b1_dx, 8)
    w2x = jnp.kron(eye8, W2_dx); b2x = jnp.tile(b2_dx, 8)
    w1z = jnp.kron(eye8, W1_dz); b1z = jnp.tile(b1_dz, 8)
    w2z = jnp.kron(eye8, W2_dz); b2z = jnp.tile(b2_dz, 8)
    full = lambda s: pl.BlockSpec(s, lambda i: (0,) * len(s))
    wp = pl.pallas_call(
        _mlp_body,
        grid=(_R8 // _BR,),
        in_specs=[
            pl.BlockSpec((_BR, 8 * A), lambda i: (i, 0)),
            full((8 * A, 8 * A)), full((8 * A,)), full((8 * A, 8)), full((8,)),
            full((8 * A, 8 * A)), full((8 * A,)), full((8 * A, 8)), full((8,)),
        ],
        out_specs=pl.BlockSpec((_BR, 8), lambda i: (i, 0)),
        out_shape=jax.ShapeDtypeStruct((_R8, 8), jnp.float32),
    )(attr8, w1x, b1x, w2x, b2x, w1z, b1z, w2z, b2z)
    return wp.reshape(EP)


_BN = 1000


def _u2_body(u, w, out):
    out[...] = jnp.tanh(jnp.dot(u[...], w[...],
                                preferred_element_type=jnp.float32))


def _u2_tc(u, W):
    return pl.pallas_call(
        _u2_body,
        grid=(N // _BN,),
        in_specs=[pl.BlockSpec((_BN, D), lambda i: (i, 0)),
                  pl.BlockSpec((D, D), lambda i: (0, 0))],
        out_specs=pl.BlockSpec((_BN, D), lambda i: (i, 0)),
        out_shape=jax.ShapeDtypeStruct((N, D), jnp.float32),
    )(u, W)


QCH = 8
CK0 = 112
CK1 = 2 * NCHUNK - CK0
TOT_CK = 2 * NCHUNK


def _sc_body(u2_hbm, dsts_hbm, srcs_hbm, ws_hbm, duA_hbm, s_hbm,
             dst_q, src_q, w_q, rows0, rows1,
             gsem0, gsem1, ssem0, ssem1, stgsem, du_sh, s_sh):
    cid = lax.axis_index("c")
    sid = lax.axis_index("s")
    wid = sid * NC + cid
    rows = (rows0, rows1)
    gsem = (gsem0, gsem1)
    ssem = (ssem0, ssem1)

    def zrow(i, _):
        for c in range(D // 16):
            rows0[i, pl.ds(c * 16, 16)] = jnp.zeros((16,), jnp.float32)
        return 0
    lax.fori_loop(0, CH, zrow, 0, unroll=4)

    base = sid * RPT
    def zdu(k, _):
        pltpu.sync_copy(rows0, du_sh.at[pl.ds(base + k * CH, CH)])
        pltpu.sync_copy(rows0.at[0], s_sh.at[pl.ds(base + k * CH, CH)])
        return 0
    lax.fori_loop(0, RPT // CH, zdu, 0)

    plsc.subcore_barrier()

    def scale(sq, c, rbuf):
        def grp(g, _):
            w16 = w_q[sq, c, pl.ds(g * 16, 16)]
            for l in range(16):
                wb = lax.broadcast_in_dim(w16[l], (16,), ())
                i = g * 16 + l
                for k in range(D // 16):
                    sl = pl.ds(k * 16, 16)
                    rbuf[i, sl] = rbuf[i, sl] * wb
            return 0
        lax.fori_loop(0, CH // 16, grp, 0)

    start = sid * TOT_CK + jnp.where(cid == 0, 0, CK0)
    nq = jnp.where(cid == 0, CK0 // QCH, CK1 // QCH)

    pltpu.sync_copy(dsts_hbm.at[pl.ds(start, QCH)], dst_q.at[0])
    pltpu.sync_copy(srcs_hbm.at[pl.ds(start, QCH)], src_q.at[0])
    pltpu.sync_copy(ws_hbm.at[pl.ds(start, QCH)], w_q.at[0])

    def round_body(q, _):
        sq = lax.rem(q, 2)
        off = start + q * QCH

        @pl.when(q > 0)
        def _():
            pltpu.make_async_copy(dsts_hbm.at[pl.ds(off, QCH)],
                                  dst_q.at[sq], stgsem).wait()
            pltpu.make_async_copy(srcs_hbm.at[pl.ds(off, QCH)],
                                  src_q.at[sq], stgsem).wait()
            pltpu.make_async_copy(ws_hbm.at[pl.ds(off, QCH)],
                                  w_q.at[sq], stgsem).wait()
            pltpu.make_async_copy(rows0, du_sh.at[src_q.at[sq, 0]],
                                  ssem0).wait()
            pltpu.make_async_copy(rows1, du_sh.at[src_q.at[sq, 0]],
                                  ssem1).wait()

        pltpu.async_copy(u2_hbm.at[dst_q.at[sq, 0]], rows0, gsem0)

        @pl.when(q + 1 < nq)
        def _():
            noff = off + QCH
            pltpu.async_copy(dsts_hbm.at[pl.ds(noff, QCH)],
                             dst_q.at[1 - sq], stgsem)
            pltpu.async_copy(srcs_hbm.at[pl.ds(noff, QCH)],
                             src_q.at[1 - sq], stgsem)
            pltpu.async_copy(ws_hbm.at[pl.ds(noff, QCH)],
                             w_q.at[1 - sq], stgsem)

        def pair(j2, _):
            for p in range(2):
                c = 2 * j2 + p
                pltpu.make_async_copy(
                    u2_hbm.at[dst_q.at[sq, c]], rows[p], gsem[p]).wait()

                @pl.when(c < QCH - 1)
                def _():
                    @pl.when(c >= 1)
                    def _():
                        pltpu.make_async_copy(
                            rows[1 - p], du_sh.at[src_q.at[sq, c]],
                            ssem[1 - p]).wait()
                    pltpu.async_copy(u2_hbm.at[dst_q.at[sq, c + 1]],
                                     rows[1 - p], gsem[1 - p])

                scale(sq, c, rows[p])
                pltpu.async_copy(rows[p], du_sh.at[src_q.at[sq, c]], ssem[p],
                                 add=True)
                pltpu.sync_copy(w_q.at[sq, c], s_sh.at[src_q.at[sq, c]],
                                add=True)
            return 0
        lax.fori_loop(0, QCH // 2, pair, 0)
        return 0
    lax.fori_loop(0, nq, round_body, 0)

    pltpu.make_async_copy(rows0, du_sh.at[src_q.at[0, 0]], ssem0).wait()
    pltpu.make_async_copy(rows1, du_sh.at[src_q.at[0, 0]], ssem1).wait()

    plsc.subcore_barrier()

    pltpu.sync_copy(du_sh.at[pl.ds(base, RPT)], duA_hbm.at[cid, pl.ds(base, RPT)])
    @pl.when(sid == 0)
    def _():
        pltpu.sync_copy(s_sh, s_hbm.at[cid])


def _sc_scatter(u2, dsts, srcs, ws):
    mesh = plsc.VectorSubcoreMesh(core_axis_name="c", subcore_axis_name="s",
                                  num_cores=NC, num_subcores=NS)
    f = pl.kernel(
        _sc_body,
        out_type=[jax.ShapeDtypeStruct((NC, NP, D), jnp.float32),
                  jax.ShapeDtypeStruct((NC, NP), jnp.float32)],
        mesh=mesh,
        scratch_types=[
            pltpu.VMEM((2, QCH, CH), jnp.int32),
            pltpu.VMEM((2, QCH, CH), jnp.int32),
            pltpu.VMEM((2, QCH, CH), jnp.float32),
            pltpu.VMEM((CH, D), jnp.float32),
            pltpu.VMEM((CH, D), jnp.float32),
            pltpu.SemaphoreType.DMA,
            pltpu.SemaphoreType.DMA,
            pltpu.SemaphoreType.DMA,
            pltpu.SemaphoreType.DMA,
            pltpu.SemaphoreType.DMA,
            pltpu.VMEM_SHARED((NP, D), jnp.float32),
            pltpu.VMEM_SHARED((NP,), jnp.float32),
        ],
    )
    return f(u2, dsts, srcs, ws)


def _final_body(u2, duA, s, out):
    du = jnp.sum(duA[...], axis=0)
    stot = jnp.sum(s[...], axis=0)
    v = du - stot * u2[...]
    out[...] = u2[...] + jnp.tanh(DELTA_T * v)


def _final_tc(u2, duA, s):
    return pl.pallas_call(
        _final_body,
        grid=(N // _BN,),
        in_specs=[pl.BlockSpec((_BN, D), lambda i: (i, 0)),
                  pl.BlockSpec((NC, _BN, D), lambda i: (0, i, 0)),
                  pl.BlockSpec((NC, _BN, 1), lambda i: (0, i, 0))],
        out_specs=pl.BlockSpec((_BN, D), lambda i: (i, 0)),
        out_shape=jax.ShapeDtypeStruct((N, D), jnp.float32),
    )(u2, duA, s.reshape(NC, NP, 1))


def kernel(u, edge_index, edge_attr, W, W1_dx, b1_dx, W2_dx, b2_dx,
           W1_dz, b1_dz, W2_dz, b2_dz):
    wp = _edge_w(edge_attr, W1_dx, b1_dx, W2_dx, b2_dx,
                 W1_dz, b1_dz, W2_dz, b2_dz)
    u2 = _u2_tc(u, W)

    pad = EP - E
    ei = jnp.concatenate(
        [edge_index, jnp.zeros((2, pad), jnp.int32)], axis=1)
    srcs = ei[0].reshape(EP // CH, CH)
    dsts = ei[1].reshape(EP // CH, CH)
    ws = wp.reshape(EP // CH, CH)

    duA, s = _sc_scatter(u2, dsts, srcs, ws)
    return _final_tc(u2, duA, s)
